# Initial kernel scaffold; baseline (speedup 1.0000x reference)
import jax
import jax.numpy as jnp
from jax.experimental import pallas as pl


def kernel(x, edge_index, batch, W1, b1, g1, be1, W2, b2, g2, be2, W3, b3, g3, be3, Wh1, bh1, Wh2, bh2):
    raise NotImplementedError("write your pallas kernel here")


# speedup vs baseline: 11.8618x; 11.8618x over previous
import functools

import jax
import jax.numpy as jnp
from jax import lax
from jax.experimental import pallas as pl
from jax.experimental.pallas import tpu as pltpu
from jax.experimental.pallas import tpu_sc as plsc

N = 50000
E = 800000
EP = 819200
ER = 6400
CHUNK = 128
SB = 16
SBD = 8
TILES = 16
ROWS_PER_TILE = ER // TILES
N_ACC = 50048
ZROWS = N_ACC // TILES
QW = 16
WO_FULL = 3200
WO_LAST = N - 15 * WO_FULL
D_IN = 128
H = 64
HH = 32
G = 128
BN_ROWS = 2000
GRID = N // BN_ROWS

@functools.cache
def _mesh():
    return plsc.VectorSubcoreMesh(core_axis_name="c", subcore_axis_name="s",
                                  num_cores=2, num_subcores=16)


def _zero_fill(zbuf, width):
    zv = jnp.zeros((16,), jnp.float32)
    for i in range(CHUNK):
        for j in range(width // 16):
            zbuf[i, pl.ds(j * 16, 16)] = zv


def _zero_acc(acc, zbuf, s, width):
    base = s * ZROWS

    def zloop(k, carry):
        pltpu.sync_copy(zbuf, acc.at[pl.ds(base + k * CHUNK, CHUNK), :])
        return carry

    lax.fori_loop(0, ZROWS // CHUNK, zloop, 0)
    rem = ZROWS % CHUNK
    pltpu.sync_copy(zbuf.at[pl.ds(0, rem), :],
                    acc.at[pl.ds(base + ZROWS - rem, rem), :])


def _writeout(acc, out_hbm, c, s, width):
    row0 = s * WO_FULL

    @pl.when(s < TILES - 1)
    def _():
        pltpu.sync_copy(acc.at[pl.ds(row0, WO_FULL), :],
                        out_hbm.at[pl.ds(c * N + row0, WO_FULL), :])

    @pl.when(s == TILES - 1)
    def _():
        pltpu.sync_copy(acc.at[pl.ds(row0, WO_LAST), :],
                        out_hbm.at[pl.ds(c * N + row0, WO_LAST), :])


def _sc_scatter_body(qbase, hq_hbm, src_hbm, dst_hbm, out_hbm,
                     sbuf, dbuf, gbuf, rows, zbuf, acc, gsem, ssem):
    c = lax.axis_index("c")
    s = lax.axis_index("s")
    _zero_fill(zbuf, QW)
    _zero_acc(acc, zbuf, s, QW)
    plsc.subcore_barrier()

    goff = (qbase + c) * N
    row0 = s * ROWS_PER_TILE

    def body(it, carry):
        r0 = row0 + it * SB
        pltpu.sync_copy(src_hbm.at[pl.ds(r0, SB), :], sbuf)
        pltpu.sync_copy(dst_hbm.at[pl.ds(r0, SB), :], dbuf)
        for k in range(SB):
            for j in range(CHUNK // 16):
                gbuf[k, pl.ds(j * 16, 16)] = (
                    sbuf[k, pl.ds(j * 16, 16)] + goff)
        gd = [pltpu.async_copy(hq_hbm.at[gbuf.at[k]], rows.at[k], gsem)
              for k in range(SB)]
        for d in gd:
            d.wait()
        sd = [pltpu.async_copy(rows.at[k], acc.at[dbuf.at[k]], ssem, add=True)
              for k in range(SB)]
        for d in sd:
            d.wait()
        return carry

    lax.fori_loop(0, ROWS_PER_TILE // SB, body, 0)
    plsc.subcore_barrier()
    _writeout(acc, out_hbm, c, s, QW)


@functools.cache
def _sc_scatter_kernel(qbase):
    return pl.kernel(
        functools.partial(_sc_scatter_body, qbase),
        mesh=_mesh(),
        compiler_params=pltpu.CompilerParams(use_tc_tiling_on_sc=False,
                                             needs_layout_passes=False),
        out_type=jax.ShapeDtypeStruct((2 * N, QW), jnp.float32),
        scratch_types=[
            pltpu.VMEM((SB, CHUNK), jnp.int32),
            pltpu.VMEM((SB, CHUNK), jnp.int32),
            pltpu.VMEM((SB, CHUNK), jnp.int32),
            pltpu.VMEM((SB, CHUNK, QW), jnp.float32),
            pltpu.VMEM((CHUNK, QW), jnp.float32),
            pltpu.VMEM_SHARED((N_ACC, QW), jnp.float32),
            pltpu.SemaphoreType.DMA,
            pltpu.SemaphoreType.DMA,
        ],
    )


def _sc_scatter(hqf, srcp, dstp):
    a = _sc_scatter_kernel(0)(hqf, srcp, dstp)
    b = _sc_scatter_kernel(2)(hqf, srcp, dstp)
    return a, b


DEG_TILE_ROWS = ER // 32


def _sc_degree_body(dst_hbm, out_hbm, dbuf, ones, zbuf, acc, ssem):
    c = lax.axis_index("c")
    s = lax.axis_index("s")
    _zero_fill(zbuf, QW)
    _zero_acc(acc, zbuf, s, QW)
    ov = jnp.ones((16,), jnp.float32)
    for i in range(CHUNK):
        ones[i, pl.ds(0, 16)] = ov
    plsc.subcore_barrier()

    row0 = c * (ER // 2) + s * DEG_TILE_ROWS

    def body(it, carry):
        pltpu.sync_copy(dst_hbm.at[pl.ds(row0 + it * SBD, SBD), :], dbuf)
        sd = [pltpu.async_copy(ones, acc.at[dbuf.at[k]], ssem, add=True)
              for k in range(SBD)]
        for d in sd:
            d.wait()
        return carry

    lax.fori_loop(0, DEG_TILE_ROWS // SBD, body, 0)
    plsc.subcore_barrier()
    _writeout(acc, out_hbm, c, s, QW)


@functools.cache
def _sc_degree_kernel():
    return pl.kernel(
        _sc_degree_body,
        mesh=_mesh(),
        compiler_params=pltpu.CompilerParams(use_tc_tiling_on_sc=False,
                                             needs_layout_passes=False),
        out_type=jax.ShapeDtypeStruct((2 * N, QW), jnp.float32),
        scratch_types=[
            pltpu.VMEM((SBD, CHUNK), jnp.int32),
            pltpu.VMEM((CHUNK, QW), jnp.float32),
            pltpu.VMEM((CHUNK, QW), jnp.float32),
            pltpu.VMEM_SHARED((N_ACC, QW), jnp.float32),
            pltpu.SemaphoreType.DMA,
        ],
    )


def _sc_degree(dstp):
    return _sc_degree_kernel()(dstp)


def _dinv_of(d0ref, d1ref):
    deg = d0ref[...] + d1ref[...] + 1.0
    return lax.rsqrt(deg)


def _write_quarters(oref, hq):
    for q in range(4):
        oref[q] = hq[:, q * QW:(q + 1) * QW]


def _k1_body(xref, wref, d0ref, d1ref, oref):
    dinv = _dinv_of(d0ref, d1ref)
    hp = jnp.dot(xref[...], wref[...], preferred_element_type=jnp.float32)
    _write_quarters(oref, hp * dinv)


def _k2_body(aggaref, aggbref, hqref, d0ref, d1ref, bref, rref, statref):
    i = pl.program_id(0)
    dinv = _dinv_of(d0ref, d1ref)
    agg = jnp.concatenate(
        [aggaref[0], aggaref[1], aggbref[0], aggbref[1]], axis=1)
    hq = jnp.concatenate([hqref[0], hqref[1], hqref[2], hqref[3]], axis=1)
    conv = dinv * (agg + hq) + bref[...]
    r = jnp.maximum(conv, 0.0)
    rref[...] = r

    @pl.when(i == 0)
    def _():
        statref[...] = jnp.zeros_like(statref)

    s0 = jnp.sum(r, axis=0, keepdims=True)
    s1 = jnp.sum(r * r, axis=0, keepdims=True)
    statref[...] += jnp.concatenate(
        [s0, s1, jnp.zeros((6, H), jnp.float32)], axis=0)


def _bn_affine(statref, gref, beref):
    m = statref[0:1, :] * (1.0 / N)
    ex2 = statref[1:2, :] * (1.0 / N)
    var = ex2 - m * m
    sinv = lax.rsqrt(var + 1e-5)
    alpha = sinv * gref[...]
    beta = beref[...] - m * alpha
    return alpha, beta


def _k3_body(rref, statref, gref, beref, wref, d0ref, d1ref, oref):
    dinv = _dinv_of(d0ref, d1ref)
    alpha, beta = _bn_affine(statref, gref, beref)
    hn = rref[...] * alpha + beta
    hp = jnp.dot(hn, wref[...], preferred_element_type=jnp.float32)
    _write_quarters(oref, hp * dinv)


def _k7_body(rref, statref, gref, beref, batchref, wh1ref, bh1ref,
             wh2ref, bh2ref, oref, psum, pcnt):
    i = pl.program_id(0)

    @pl.when(i == 0)
    def _():
        psum[...] = jnp.zeros_like(psum)
        pcnt[...] = jnp.zeros_like(pcnt)

    alpha, beta = _bn_affine(statref, gref, beref)
    h3 = rref[...] * alpha + beta
    b = batchref[...]
    onehot = (b == lax.broadcasted_iota(jnp.int32, (BN_ROWS, G), 1)
              ).astype(jnp.float32)
    dn = (((0,), (0,)), ((), ()))
    psum[...] += lax.dot_general(onehot, h3, dn,
                                 precision=lax.Precision.HIGHEST,
                                 preferred_element_type=jnp.float32)
    pcnt[...] += lax.dot_general(onehot, jnp.ones((BN_ROWS, 8), jnp.float32),
                                 dn, preferred_element_type=jnp.float32)

    @pl.when(i == GRID - 1)
    def _():
        pooled = psum[...] / jnp.maximum(pcnt[:, 0:1], 1.0)
        z = jnp.maximum(
            jnp.dot(pooled, wh1ref[...], preferred_element_type=jnp.float32)
            + bh1ref[...], 0.0)
        wh2b = jnp.broadcast_to(wh2ref[...], (HH, 8))
        oref[...] = (jnp.dot(z, wh2b, preferred_element_type=jnp.float32)
                     + bh2ref[...])


def _full(shape):
    return pl.BlockSpec(shape, lambda i: (0,) * len(shape))


_ROWB = pl.BlockSpec((BN_ROWS, H), lambda i: (i, 0))
_DEGB = pl.BlockSpec((BN_ROWS, 1), lambda i: (i, 0))
_QTRS = pl.BlockSpec((4, BN_ROWS, QW), lambda i: (0, i, 0))
_AGGB = pl.BlockSpec((2, BN_ROWS, QW), lambda i: (0, i, 0))
_STATB = pl.BlockSpec((8, H), lambda i: (0, 0))


def _k1(x, w, d0, d1):
    return pl.pallas_call(
        _k1_body,
        grid=(GRID,),
        in_specs=[pl.BlockSpec((BN_ROWS, D_IN), lambda i: (i, 0)),
                  _full((D_IN, H)), _DEGB, _DEGB],
        out_specs=_QTRS,
        out_shape=jax.ShapeDtypeStruct((4, N, QW), jnp.float32),
    )(x, w, d0, d1)


def _k2(agga, aggb, hq4, d0, d1, b):
    return pl.pallas_call(
        _k2_body,
        grid=(GRID,),
        in_specs=[_AGGB, _AGGB, _QTRS, _DEGB, _DEGB, _full((1, H))],
        out_specs=[_ROWB, _STATB],
        out_shape=[jax.ShapeDtypeStruct((N, H), jnp.float32),
                   jax.ShapeDtypeStruct((8, H), jnp.float32)],
    )(agga, aggb, hq4, d0, d1, b)


def _k3(r, stat, g, be, w, d0, d1):
    return pl.pallas_call(
        _k3_body,
        grid=(GRID,),
        in_specs=[_ROWB, _STATB, _full((1, H)), _full((1, H)),
                  _full((H, H)), _DEGB, _DEGB],
        out_specs=_QTRS,
        out_shape=jax.ShapeDtypeStruct((4, N, QW), jnp.float32),
    )(r, stat, g, be, w, d0, d1)


def _k7(r, stat, g, be, batch2, wh1, bh1, wh2, bh2):
    return pl.pallas_call(
        _k7_body,
        grid=(GRID,),
        in_specs=[_ROWB, _STATB, _full((1, H)), _full((1, H)),
                  pl.BlockSpec((BN_ROWS, 1), lambda i: (i, 0)),
                  _full((H, HH)), _full((1, HH)), _full((HH, 1)),
                  _full((1, 1))],
        out_specs=_full((G, 8)),
        out_shape=jax.ShapeDtypeStruct((G, 8), jnp.float32),
        scratch_shapes=[pltpu.VMEM((G, H), jnp.float32),
                        pltpu.VMEM((G, 8), jnp.float32)],
    )(r, stat, g, be, batch2, wh1, bh1, wh2, bh2)


def kernel(x, edge_index, batch, W1, b1, g1, be1, W2, b2, g2, be2,
           W3, b3, g3, be3, Wh1, bh1, Wh2, bh2):
    src = edge_index[0]
    dst = edge_index[1]
    pad = EP - E
    srcp = jnp.concatenate(
        [src, jnp.zeros((pad,), jnp.int32)]).reshape(ER, CHUNK)
    dstp = jnp.concatenate(
        [dst, jnp.full((pad,), N, jnp.int32)]).reshape(ER, CHUNK)

    degp = _sc_degree(dstp)
    d0 = degp[:N, 0:1]
    d1 = degp[N:, 0:1]

    b1r, b2r, b3r = (v.reshape(1, H) for v in (b1, b2, b3))
    g1r, g2r, g3r = (v.reshape(1, H) for v in (g1, g2, g3))
    be1r, be2r, be3r = (v.reshape(1, H) for v in (be1, be2, be3))

    hq = _k1(x, W1, d0, d1)
    agga, aggb = _sc_scatter(hq.reshape(4 * N, QW), srcp, dstp)
    r1, st1 = _k2(agga.reshape(2, N, QW), aggb.reshape(2, N, QW),
                  hq, d0, d1, b1r)

    hq = _k3(r1, st1, g1r, be1r, W2, d0, d1)
    agga, aggb = _sc_scatter(hq.reshape(4 * N, QW), srcp, dstp)
    r2, st2 = _k2(agga.reshape(2, N, QW), aggb.reshape(2, N, QW),
                  hq, d0, d1, b2r)

    hq = _k3(r2, st2, g2r, be2r, W3, d0, d1)
    agga, aggb = _sc_scatter(hq.reshape(4 * N, QW), srcp, dstp)
    r3, st3 = _k2(agga.reshape(2, N, QW), aggb.reshape(2, N, QW),
                  hq, d0, d1, b3r)

    o8 = _k7(r3, st3, g3r, be3r, batch.reshape(N, 1),
             Wh1, bh1.reshape(1, HH), Wh2, bh2.reshape(1, 1))
    return o8[:, 0]


# speedup vs baseline: 14.4501x; 1.2182x over previous
import functools

import jax
import jax.numpy as jnp
from jax import lax
from jax.experimental import pallas as pl
from jax.experimental.pallas import tpu as pltpu
from jax.experimental.pallas import tpu_sc as plsc

N = 50000
E = 800000
EP = 819200
ER = 6400
CHUNK = 128
SB = 8
SBD = 8
TILES = 16
ROWS_PER_TILE = ER // TILES
N_ACC = 50048
ZROWS = N_ACC // TILES
QW = 16
WO_FULL = 3200
WO_LAST = N - 15 * WO_FULL
D_IN = 128
H = 64
G = 128
NP = N // 2
BP = 1000
GRID = NP // BP


@functools.cache
def _mesh():
    return plsc.VectorSubcoreMesh(core_axis_name="c", subcore_axis_name="s",
                                  num_cores=2, num_subcores=16)


def _zero_fill(zbuf, width):
    zv = jnp.zeros((16,), jnp.float32)
    for i in range(CHUNK):
        for j in range(width // 16):
            zbuf[i, pl.ds(j * 16, 16)] = zv


def _zero_acc(acc, zbuf, s, width):
    base = s * ZROWS

    def zloop(k, carry):
        pltpu.sync_copy(zbuf, acc.at[pl.ds(base + k * CHUNK, CHUNK), :])
        return carry

    lax.fori_loop(0, ZROWS // CHUNK, zloop, 0)
    rem = ZROWS % CHUNK
    pltpu.sync_copy(zbuf.at[pl.ds(0, rem), :],
                    acc.at[pl.ds(base + ZROWS - rem, rem), :])


def _sc_scatter_body(qbase, hq_hbm, src_hbm, dst_hbm, out_hbm,
                     sbufA, sbufB, dbufA, dbufB, gbufA, gbufB,
                     rowsA, rowsB, zbuf, acc, gsemA, gsemB, ssem):
    c = lax.axis_index("c")
    s = lax.axis_index("s")

    def run(q):
        _zero_fill(zbuf, QW)
        _zero_acc(acc, zbuf, s, QW)
        plsc.subcore_barrier()
        row0 = s * ROWS_PER_TILE

        def halfburst(r0, sbuf, dbuf, gbuf, rows, gsem):
            pltpu.sync_copy(src_hbm.at[pl.ds(r0, SB), :], sbuf)
            pltpu.sync_copy(dst_hbm.at[pl.ds(r0, SB), :], dbuf)
            for k in range(SB):
                for j in range(CHUNK // 16):
                    v = sbuf[k, pl.ds(j * 16, 16)]
                    gbuf[k, pl.ds(j * 16, 16)] = v * 4 + q
            return [pltpu.async_copy(hq_hbm.at[gbuf.at[k]], rows.at[k], gsem)
                    for k in range(SB)]

        def drain_scatters(n):
            for _ in range(n):
                pltpu.make_async_copy(
                    hq_hbm.at[pl.ds(0, CHUNK), :], rowsA.at[0], ssem).wait()

        def body(t, carry):
            r0 = row0 + t * 2 * SB
            gdA = halfburst(r0, sbufA, dbufA, gbufA, rowsA, gsemA)
            gdB = halfburst(r0 + SB, sbufB, dbufB, gbufB, rowsB, gsemB)

            @pl.when(t > 0)
            def _():
                drain_scatters(2 * SB)

            for d in gdA:
                d.wait()
            for k in range(SB):
                pltpu.async_copy(rowsA.at[k], acc.at[dbufA.at[k]], ssem,
                                 add=True)
            for d in gdB:
                d.wait()
            for k in range(SB):
                pltpu.async_copy(rowsB.at[k], acc.at[dbufB.at[k]], ssem,
                                 add=True)
            return carry

        lax.fori_loop(0, ROWS_PER_TILE // (2 * SB), body, 0)
        drain_scatters(2 * SB)
        plsc.subcore_barrier()
        row0w = s * WO_FULL

        @pl.when(s < TILES - 1)
        def _():
            pltpu.sync_copy(acc.at[pl.ds(row0w, WO_FULL), :],
                            out_hbm.at[pl.ds(row0w, WO_FULL),
                                       pl.ds(q * QW, QW)])

        @pl.when(s == TILES - 1)
        def _():
            pltpu.sync_copy(acc.at[pl.ds(row0w, WO_LAST), :],
                            out_hbm.at[pl.ds(row0w, WO_LAST),
                                       pl.ds(q * QW, QW)])

    @pl.when(c == 0)
    def _():
        run(qbase)

    @pl.when(c == 1)
    def _():
        run(qbase + 1)


@functools.cache
def _sc_scatter_kernel(qbase):
    return pl.kernel(
        functools.partial(_sc_scatter_body, qbase),
        mesh=_mesh(),
        compiler_params=pltpu.CompilerParams(use_tc_tiling_on_sc=False,
                                             needs_layout_passes=False),
        out_type=jax.ShapeDtypeStruct((N, H), jnp.float32),
        scratch_types=[
            pltpu.VMEM((SB, CHUNK), jnp.int32),
            pltpu.VMEM((SB, CHUNK), jnp.int32),
            pltpu.VMEM((SB, CHUNK), jnp.int32),
            pltpu.VMEM((SB, CHUNK), jnp.int32),
            pltpu.VMEM((SB, CHUNK), jnp.int32),
            pltpu.VMEM((SB, CHUNK), jnp.int32),
            pltpu.VMEM((SB, CHUNK, QW), jnp.float32),
            pltpu.VMEM((SB, CHUNK, QW), jnp.float32),
            pltpu.VMEM((CHUNK, QW), jnp.float32),
            pltpu.VMEM_SHARED((N_ACC, QW), jnp.float32),
            pltpu.SemaphoreType.DMA,
            pltpu.SemaphoreType.DMA,
            pltpu.SemaphoreType.DMA,
        ],
    )


def _sc_scatter(hq4, srcp, dstp):
    a = _sc_scatter_kernel(0)(hq4, srcp, dstp)
    b = _sc_scatter_kernel(2)(hq4, srcp, dstp)
    return a, b


DEG_TILE_ROWS = ER // 32


def _sc_degree_body(dst_hbm, out_hbm, dbuf, ones, zbuf, acc, ssem):
    c = lax.axis_index("c")
    s = lax.axis_index("s")
    _zero_fill(zbuf, QW)
    _zero_acc(acc, zbuf, s, QW)
    ov = jnp.ones((16,), jnp.float32)
    for i in range(CHUNK):
        ones[i, pl.ds(0, 16)] = ov
    plsc.subcore_barrier()

    row0 = c * (ER // 2) + s * DEG_TILE_ROWS

    def body(it, carry):
        pltpu.sync_copy(dst_hbm.at[pl.ds(row0 + it * SBD, SBD), :], dbuf)
        sd = [pltpu.async_copy(ones, acc.at[dbuf.at[k]], ssem, add=True)
              for k in range(SBD)]
        for d in sd:
            d.wait()
        return carry

    lax.fori_loop(0, DEG_TILE_ROWS // SBD, body, 0)
    plsc.subcore_barrier()
    row0w = s * WO_FULL

    def wout(n):
        pltpu.sync_copy(acc.at[pl.ds(row0w, n), :],
                        out_hbm.at[pl.ds(c * N + row0w, n), :])

    @pl.when(s < TILES - 1)
    def _():
        wout(WO_FULL)

    @pl.when(s == TILES - 1)
    def _():
        wout(WO_LAST)


@functools.cache
def _sc_degree_kernel():
    return pl.kernel(
        _sc_degree_body,
        mesh=_mesh(),
        compiler_params=pltpu.CompilerParams(use_tc_tiling_on_sc=False,
                                             needs_layout_passes=False),
        out_type=jax.ShapeDtypeStruct((2 * N, QW), jnp.float32),
        scratch_types=[
            pltpu.VMEM((SBD, CHUNK), jnp.int32),
            pltpu.VMEM((CHUNK, QW), jnp.float32),
            pltpu.VMEM((CHUNK, QW), jnp.float32),
            pltpu.VMEM_SHARED((N_ACC, QW), jnp.float32),
            pltpu.SemaphoreType.DMA,
        ],
    )


def _sc_degree(dstp):
    return _sc_degree_kernel()(dstp)


def _k1_body(xref, wref, ddref, oref):
    dinv = lax.rsqrt(ddref[...])
    hp = jnp.dot(xref[...], wref[...], preferred_element_type=jnp.float32)
    oref[...] = hp * dinv


def _mix(aref, bref):
    a = aref[...]
    b = bref[...]
    return jnp.concatenate([a[:, 0:32], b[:, 32:64],
                            a[:, 64:96], b[:, 96:128]], axis=1)


def _k2_body(aggaref, aggbref, hqref, ddref, bref, rref, statref):
    i = pl.program_id(0)
    dinv = lax.rsqrt(ddref[...])
    conv = dinv * (_mix(aggaref, aggbref) + hqref[...]) + bref[...]
    r = jnp.maximum(conv, 0.0)
    rref[...] = r

    @pl.when(i == 0)
    def _():
        statref[...] = jnp.zeros_like(statref)

    s0 = jnp.sum(r, axis=0, keepdims=True)
    s1 = jnp.sum(r * r, axis=0, keepdims=True)
    statref[...] += jnp.concatenate(
        [s0, s1, jnp.zeros((6, 2 * H), jnp.float32)], axis=0)


def _bn_affine(statref, gref, beref):
    sm = statref[0:1, 0:H] + statref[0:1, H:2 * H]
    sq = statref[1:2, 0:H] + statref[1:2, H:2 * H]
    m = sm * (1.0 / N)
    var = sq * (1.0 / N) - m * m
    sinv = lax.rsqrt(var + 1e-5)
    alpha = sinv * gref[...]
    beta = beref[...] - m * alpha
    a2 = jnp.concatenate([alpha, alpha], axis=1)
    b2 = jnp.concatenate([beta, beta], axis=1)
    return a2, b2


def _k3_body(rref, statref, gref, beref, wref, ddref, oref):
    dinv = lax.rsqrt(ddref[...])
    a2, b2 = _bn_affine(statref, gref, beref)
    hn = rref[...] * a2 + b2
    hp = jnp.dot(hn, wref[...], preferred_element_type=jnp.float32)
    oref[...] = hp * dinv


def _k7_body(rref, statref, gref, beref, bEref, bOref, wh1ref, bh1ref,
             wh2ref, bh2ref, oref, psum, pcnt):
    i = pl.program_id(0)

    @pl.when(i == 0)
    def _():
        psum[...] = jnp.zeros_like(psum)
        pcnt[...] = jnp.zeros_like(pcnt)

    a2, b2 = _bn_affine(statref, gref, beref)
    h3 = rref[...] * a2 + b2
    lanes = lax.broadcasted_iota(jnp.int32, (BP, G), 1)
    ohE = (bEref[...] == lanes).astype(jnp.float32)
    ohO = (bOref[...] == lanes).astype(jnp.float32)
    dn = (((0,), (0,)), ((), ()))
    hi = lax.Precision.HIGHEST
    psum[...] += lax.dot_general(ohE, h3[:, 0:H], dn, precision=hi,
                                 preferred_element_type=jnp.float32)
    psum[...] += lax.dot_general(ohO, h3[:, H:2 * H], dn, precision=hi,
                                 preferred_element_type=jnp.float32)
    pcnt[...] += lax.dot_general(ohE + ohO, jnp.ones((BP, 8), jnp.float32),
                                 dn, preferred_element_type=jnp.float32)

    @pl.when(i == GRID - 1)
    def _():
        pooled = psum[...] / jnp.maximum(pcnt[:, 0:1], 1.0)
        z = jnp.maximum(
            jnp.dot(pooled, wh1ref[...], preferred_element_type=jnp.float32)
            + bh1ref[...], 0.0)
        wh2b = jnp.broadcast_to(wh2ref[...], (H // 2, 8))
        oref[...] = (jnp.dot(z, wh2b, preferred_element_type=jnp.float32)
                     + bh2ref[...])


def _full(shape):
    return pl.BlockSpec(shape, lambda i: (0,) * len(shape))


_PAIRB = pl.BlockSpec((BP, 2 * H), lambda i: (i, 0))
_STATB = pl.BlockSpec((8, 2 * H), lambda i: (0, 0))


def _k1(x2, w2, dd):
    return pl.pallas_call(
        _k1_body,
        grid=(GRID,),
        in_specs=[pl.BlockSpec((BP, 2 * D_IN), lambda i: (i, 0)),
                  _full((2 * D_IN, 2 * H)), _PAIRB],
        out_specs=_PAIRB,
        out_shape=jax.ShapeDtypeStruct((NP, 2 * H), jnp.float32),
    )(x2, w2, dd)


def _k2(agga, aggb, hq, dd, b):
    return pl.pallas_call(
        _k2_body,
        grid=(GRID,),
        in_specs=[_PAIRB, _PAIRB, _PAIRB, _PAIRB, _full((1, 2 * H))],
        out_specs=[_PAIRB, _STATB],
        out_shape=[jax.ShapeDtypeStruct((NP, 2 * H), jnp.float32),
                   jax.ShapeDtypeStruct((8, 2 * H), jnp.float32)],
    )(agga, aggb, hq, dd, b)


def _k3(r, stat, g, be, w2, dd):
    return pl.pallas_call(
        _k3_body,
        grid=(GRID,),
        in_specs=[_PAIRB, _STATB, _full((1, H)), _full((1, H)),
                  _full((2 * H, 2 * H)), _PAIRB],
        out_specs=_PAIRB,
        out_shape=jax.ShapeDtypeStruct((NP, 2 * H), jnp.float32),
    )(r, stat, g, be, w2, dd)


def _k7(r, stat, g, be, bE, bO, wh1, bh1, wh2, bh2):
    return pl.pallas_call(
        _k7_body,
        grid=(GRID,),
        in_specs=[_PAIRB, _STATB, _full((1, H)), _full((1, H)),
                  pl.BlockSpec((BP, G), lambda i: (i, 0)),
                  pl.BlockSpec((BP, G), lambda i: (i, 0)),
                  _full((H, H // 2)), _full((1, H // 2)),
                  _full((H // 2, 1)), _full((1, 1))],
        out_specs=_full((G, 8)),
        out_shape=jax.ShapeDtypeStruct((G, 8), jnp.float32),
        scratch_shapes=[pltpu.VMEM((G, H), jnp.float32),
                        pltpu.VMEM((G, 8), jnp.float32)],
    )(r, stat, g, be, bE, bO, wh1, bh1, wh2, bh2)


def _blockdiag2(w):
    d, h = w.shape
    z = jnp.zeros((d, h), jnp.float32)
    top = jnp.concatenate([w, z], axis=1)
    bot = jnp.concatenate([z, w], axis=1)
    return jnp.concatenate([top, bot], axis=0)


def kernel(x, edge_index, batch, W1, b1, g1, be1, W2, b2, g2, be2,
           W3, b3, g3, be3, Wh1, bh1, Wh2, bh2):
    src = edge_index[0]
    dst = edge_index[1]
    pad = EP - E
    srcp = jnp.concatenate(
        [src, jnp.zeros((pad,), jnp.int32)]).reshape(ER, CHUNK)
    dstp = jnp.concatenate(
        [dst, jnp.full((pad,), N, jnp.int32)]).reshape(ER, CHUNK)

    degp = _sc_degree(dstp)
    degsum = degp[:N, 0] + degp[N:, 0] + 1.0
    dd = jnp.concatenate(
        [jnp.broadcast_to(degsum[0::2, None], (NP, H)),
         jnp.broadcast_to(degsum[1::2, None], (NP, H))], axis=1)

    bE = jnp.broadcast_to(batch[0::2, None], (NP, G))
    bO = jnp.broadcast_to(batch[1::2, None], (NP, G))

    x2 = x.reshape(NP, 2 * D_IN)
    W1x = _blockdiag2(W1)
    W2x = _blockdiag2(W2)
    W3x = _blockdiag2(W3)
    b1r, b2r, b3r = (jnp.concatenate([v, v]).reshape(1, 2 * H)
                     for v in (b1, b2, b3))
    g1r, g2r, g3r = (v.reshape(1, H) for v in (g1, g2, g3))
    be1r, be2r, be3r = (v.reshape(1, H) for v in (be1, be2, be3))

    def qview(a):
        return a.reshape(4 * N, QW)

    def pview(a):
        return a.reshape(NP, 2 * H)

    hq = _k1(x2, W1x, dd)
    agga, aggb = _sc_scatter(qview(hq), srcp, dstp)
    r1, st1 = _k2(pview(agga), pview(aggb), hq, dd, b1r)

    hq = _k3(r1, st1, g1r, be1r, W2x, dd)
    agga, aggb = _sc_scatter(qview(hq), srcp, dstp)
    r2, st2 = _k2(pview(agga), pview(aggb), hq, dd, b2r)

    hq = _k3(r2, st2, g2r, be2r, W3x, dd)
    agga, aggb = _sc_scatter(qview(hq), srcp, dstp)
    r3, st3 = _k2(pview(agga), pview(aggb), hq, dd, b3r)

    o8 = _k7(r3, st3, g3r, be3r, bE, bO,
             Wh1, bh1.reshape(1, H // 2), Wh2, bh2.reshape(1, 1))
    return o8[:, 0]
